# baseline (speedup 1.0000x reference)
import functools

import jax
import jax.numpy as jnp
from jax import lax
from jax.experimental import pallas as pl
from jax.experimental.pallas import tpu as pltpu
from jax.experimental.pallas import tpu_sc as plsc

B, S, D = 4, 2048, 1024
V, H = 128, 1024
TOTAL = S + V
NW = 32
WPB = NW // B
CHUNK = TOTAL // WPB
ZPAD = CHUNK + 8
AUXR = V + ZPAD

_POWS = (256, 128, 64, 32, 16, 8, 4, 2, 1)


def _mlp_body(w_emb_ref, w1_ref, b1_ref, w2_ref, b2_ref, aux_ref):
    h = jnp.tanh(
        jnp.dot(w_emb_ref[...], w1_ref[...], preferred_element_type=jnp.float32)
        + b1_ref[...]
    )
    virt = (
        jnp.dot(h, w2_ref[...], preferred_element_type=jnp.float32) + b2_ref[...]
    )
    aux_ref[0:V, :] = virt
    aux_ref[V:, :] = jnp.zeros((ZPAD, D), jnp.float32)


def _make_aux(w_emb, w1, b1, w2, b2):
    return pl.pallas_call(
        _mlp_body,
        out_shape=jax.ShapeDtypeStruct((AUXR, D), jnp.float32),
    )(w_emb, w1, b1.reshape(1, H), w2, b2.reshape(1, D))


def _copy_region(src_ref, dst_ref, src_base, dst_base, count, max_pow):
    for p in _POWS:
        if p > max_pow:
            continue
        off = jnp.bitwise_and(count, jnp.int32(~(2 * p - 1)))

        @pl.when(jnp.bitwise_and(count, jnp.int32(p)) != 0)
        def _():
            pltpu.sync_copy(
                src_ref.at[pl.ds((src_base + off) * D, p * D)],
                dst_ref.at[pl.ds((dst_base + off) * D, p * D)],
            )


@functools.partial(
    pl.kernel,
    mesh=plsc.VectorSubcoreMesh(core_axis_name="c", subcore_axis_name="s"),
    out_type=jax.ShapeDtypeStruct((B * TOTAL * D,), jnp.float32),
    scratch_types=[pltpu.VMEM((32,), jnp.int32)],
)
def _sc_union(inp_hbm, seq_hbm, aux_hbm, out_hbm, seq_v):
    cid = lax.axis_index("c")
    sid = lax.axis_index("s")
    w = sid * 2 + cid
    b = w // WPB
    t0 = (w % WPB) * CHUNK

    pltpu.sync_copy(seq_hbm, seq_v)
    ln = seq_v[pl.ds(b, 16)][0]

    out_base = b * TOTAL

    k1 = jnp.clip(ln - t0, 0, CHUNK)
    _copy_region(inp_hbm, out_hbm, b * S + t0, out_base + t0, k1, 256)

    s2 = jnp.clip(ln, t0, t0 + CHUNK)
    e2 = jnp.clip(ln + V, t0, t0 + CHUNK)
    k2 = e2 - s2
    _copy_region(aux_hbm, out_hbm, jnp.maximum(s2 - ln, 0), out_base + s2, k2, 128)

    s3 = jnp.clip(ln + V, t0, t0 + CHUNK)
    k3 = t0 + CHUNK - s3
    _copy_region(aux_hbm, out_hbm, jnp.int32(V), out_base + s3, k3, 256)


def kernel(inputs, seq_len, W_emb, W1, b1, W2, b2):
    seq32 = seq_len.astype(jnp.int32)
    aux = _make_aux(W_emb, W1, b1, W2, b2)
    seq_pad = jnp.zeros((32,), jnp.int32).at[:B].set(seq32)
    out_flat = _sc_union(inputs.reshape(B * S * D), seq_pad, aux.reshape(AUXR * D))
    return out_flat.reshape(B, TOTAL, D), seq_len + V


# speedup vs baseline: 1.0007x; 1.0007x over previous
import functools

import jax
import jax.numpy as jnp
from jax import lax
from jax.experimental import pallas as pl
from jax.experimental.pallas import tpu as pltpu
from jax.experimental.pallas import tpu_sc as plsc

B, S, D = 4, 2048, 1024
V, H = 128, 1024
TOTAL = S + V
NW = 32
WPB = NW // B
CHUNK = TOTAL // WPB
ZPAD = CHUNK + 8
AUXR = V + ZPAD

_POWS = (256, 128, 64, 32, 16, 8, 4, 2, 1)


def _mlp_body(w_emb_ref, w1_ref, b1_ref, w2_ref, b2_ref, aux_ref):
    h = jnp.tanh(
        jnp.dot(w_emb_ref[...], w1_ref[...], preferred_element_type=jnp.float32)
        + b1_ref[...]
    )
    virt = (
        jnp.dot(h, w2_ref[...], preferred_element_type=jnp.float32) + b2_ref[...]
    )
    aux_ref[0:V, :] = virt
    aux_ref[V:, :] = jnp.zeros((ZPAD, D), jnp.float32)


def _make_aux(w_emb, w1, b1, w2, b2):
    return pl.pallas_call(
        _mlp_body,
        out_shape=jax.ShapeDtypeStruct((AUXR, D), jnp.float32),
    )(w_emb, w1, b1.reshape(1, H), w2, b2.reshape(1, D))


def _copy_region(src_ref, dst_ref, src_base, dst_base, count, max_pow, sem):
    for p in _POWS:
        if p > max_pow:
            continue
        off = jnp.bitwise_and(count, jnp.int32(~(2 * p - 1)))

        @pl.when(jnp.bitwise_and(count, jnp.int32(p)) != 0)
        def _():
            pltpu.async_copy(
                src_ref.at[pl.ds((src_base + off) * D, p * D)],
                dst_ref.at[pl.ds((dst_base + off) * D, p * D)],
                sem,
            )


@functools.partial(
    pl.kernel,
    mesh=plsc.VectorSubcoreMesh(core_axis_name="c", subcore_axis_name="s"),
    out_type=jax.ShapeDtypeStruct((B * TOTAL * D,), jnp.float32),
    scratch_types=[pltpu.VMEM((32,), jnp.int32), pltpu.SemaphoreType.DMA],
)
def _sc_union(inp_hbm, seq_hbm, aux_hbm, out_hbm, seq_v, sem):
    cid = lax.axis_index("c")
    sid = lax.axis_index("s")
    w = sid * 2 + cid
    b = w // WPB
    t0 = (w % WPB) * CHUNK

    pltpu.sync_copy(seq_hbm, seq_v)
    ln = seq_v[pl.ds(b, 16)][0]

    out_base = b * TOTAL

    k1 = jnp.clip(ln - t0, 0, CHUNK)
    _copy_region(inp_hbm, out_hbm, b * S + t0, out_base + t0, k1, 256, sem)

    s2 = jnp.clip(ln, t0, t0 + CHUNK)
    e2 = jnp.clip(ln + V, t0, t0 + CHUNK)
    k2 = e2 - s2
    _copy_region(aux_hbm, out_hbm, jnp.maximum(s2 - ln, 0), out_base + s2, k2, 128, sem)

    s3 = jnp.clip(ln + V, t0, t0 + CHUNK)
    k3 = t0 + CHUNK - s3
    _copy_region(aux_hbm, out_hbm, jnp.int32(V), out_base + s3, k3, 256, sem)

    drain = out_hbm.at[pl.ds((out_base + t0) * D, CHUNK * D)]
    pltpu.make_async_copy(drain, drain, sem).wait()


def kernel(inputs, seq_len, W_emb, W1, b1, W2, b2):
    seq32 = seq_len.astype(jnp.int32)
    aux = _make_aux(W_emb, W1, b1, W2, b2)
    seq_pad = jnp.zeros((32,), jnp.int32).at[:B].set(seq32)
    out_flat = _sc_union(inputs.reshape(B * S * D), seq_pad, aux.reshape(AUXR * D))
    return out_flat.reshape(B, TOTAL, D), seq_len + V


# speedup vs baseline: 9.6259x; 9.6188x over previous
import functools

import jax
import jax.numpy as jnp
from jax import lax
from jax.experimental import pallas as pl
from jax.experimental.pallas import tpu as pltpu
from jax.experimental.pallas import tpu_sc as plsc

B, S, D = 4, 2048, 1024
V, H = 128, 1024
TOTAL = S + V
NW = 32
WPB = NW // B
CHUNK = TOTAL // WPB
ZPAD = CHUNK + 8
AUXR = V + ZPAD

def _mlp_body(w_emb_ref, w1_ref, b1_ref, w2_ref, b2_ref, aux_ref):
    h = jnp.tanh(
        jnp.dot(w_emb_ref[...], w1_ref[...], preferred_element_type=jnp.float32)
        + b1_ref[...]
    )
    virt = (
        jnp.dot(h, w2_ref[...], preferred_element_type=jnp.float32) + b2_ref[...]
    )
    aux_ref[0:V, :] = virt
    aux_ref[V:, :] = jnp.zeros((ZPAD, D), jnp.float32)


def _make_aux(w_emb, w1, b1, w2, b2):
    return pl.pallas_call(
        _mlp_body,
        out_shape=jax.ShapeDtypeStruct((AUXR, D), jnp.float32),
    )(w_emb, w1, b1.reshape(1, H), w2, b2.reshape(1, D))


CAP = 96


def _piece(src_ref, dst_ref, buf, src_base, dst_base, off, rows):
    pltpu.sync_copy(src_ref.at[pl.ds((src_base + off) * D, rows * D)],
                    buf.at[pl.ds(0, rows * D)])
    pltpu.sync_copy(buf.at[pl.ds(0, rows * D)],
                    dst_ref.at[pl.ds((dst_base + off) * D, rows * D)])


def _copy_region(src_ref, dst_ref, buf, src_base, dst_base, count, max_count):
    nblk = (max_count + CAP - 1) // CAP
    for i in range(nblk):
        @pl.when(count >= (i + 1) * CAP)
        def _():
            _piece(src_ref, dst_ref, buf, src_base, dst_base, i * CAP, CAP)
    blkrows = (count // CAP) * CAP
    rem = count - blkrows
    for p in (64, 32, 16, 8, 4, 2, 1):
        if p >= CAP or p > max_count:
            continue
        off = blkrows + jnp.bitwise_and(rem, jnp.int32(~(2 * p - 1)))

        @pl.when(jnp.bitwise_and(rem, jnp.int32(p)) != 0)
        def _():
            _piece(src_ref, dst_ref, buf, src_base, dst_base, off, p)


@functools.partial(
    pl.kernel,
    mesh=plsc.VectorSubcoreMesh(core_axis_name="c", subcore_axis_name="s"),
    out_type=jax.ShapeDtypeStruct((B * TOTAL * D,), jnp.float32),
    scratch_types=[pltpu.VMEM((32,), jnp.int32), pltpu.VMEM((CAP * D,), jnp.float32)],
)
def _sc_union(inp_hbm, seq_hbm, aux_hbm, out_hbm, seq_v, buf):
    cid = lax.axis_index("c")
    sid = lax.axis_index("s")
    w = sid * 2 + cid
    b = w // WPB
    t0 = (w % WPB) * CHUNK

    pltpu.sync_copy(seq_hbm, seq_v)
    ln = seq_v[pl.ds(b, 16)][0]

    out_base = b * TOTAL

    k1 = jnp.clip(ln - t0, 0, CHUNK)
    _copy_region(inp_hbm, out_hbm, buf, b * S + t0, out_base + t0, k1, CHUNK)

    s2 = jnp.clip(ln, t0, t0 + CHUNK)
    e2 = jnp.clip(ln + V, t0, t0 + CHUNK)
    k2 = e2 - s2
    _copy_region(aux_hbm, out_hbm, buf, jnp.maximum(s2 - ln, 0), out_base + s2, k2, V)

    s3 = jnp.clip(ln + V, t0, t0 + CHUNK)
    k3 = t0 + CHUNK - s3
    _copy_region(aux_hbm, out_hbm, buf, jnp.int32(V), out_base + s3, k3, CHUNK)


def kernel(inputs, seq_len, W_emb, W1, b1, W2, b2):
    seq32 = seq_len.astype(jnp.int32)
    aux = _make_aux(W_emb, W1, b1, W2, b2)
    seq_pad = jnp.zeros((32,), jnp.int32).at[:B].set(seq32)
    out_flat = _sc_union(inputs.reshape(B * S * D), seq_pad, aux.reshape(AUXR * D))
    return out_flat.reshape(B, TOTAL, D), seq_len + V


# speedup vs baseline: 19.1281x; 1.9871x over previous
import functools

import jax
import jax.numpy as jnp
from jax import lax
from jax.experimental import pallas as pl
from jax.experimental.pallas import tpu as pltpu
from jax.experimental.pallas import tpu_sc as plsc

B, S, D = 4, 2048, 1024
V, H = 128, 1024
TOTAL = S + V
NW = 32
WPB = NW // B
CHUNK = TOTAL // WPB
ZPAD = CHUNK + 8
AUXR = 8 + V + ZPAD
CAP = 96


def _mlp_body(seq_ref, w_emb_ref, w1_ref, b1_ref, w2_ref, b2_ref, aux_ref):
    h = jnp.tanh(
        jnp.dot(w_emb_ref[...], w1_ref[...], preferred_element_type=jnp.float32)
        + b1_ref[...]
    )
    virt = (
        jnp.dot(h, w2_ref[...], preferred_element_type=jnp.float32) + b2_ref[...]
    )
    aux_ref[...] = jnp.zeros((B, AUXR, D), jnp.float32)
    for b in range(B):
        rb = seq_ref[b] % 8
        for s in range(8):
            @pl.when(rb == s)
            def _():
                aux_ref[b, s:s + V, :] = virt


def _make_aux(seq32, w_emb, w1, b1, w2, b2):
    return pl.pallas_call(
        _mlp_body,
        out_shape=jax.ShapeDtypeStruct((B, AUXR, D), jnp.float32),
        in_specs=[
            pl.BlockSpec(memory_space=pltpu.SMEM),
            pl.BlockSpec(memory_space=pltpu.VMEM),
            pl.BlockSpec(memory_space=pltpu.VMEM),
            pl.BlockSpec(memory_space=pltpu.VMEM),
            pl.BlockSpec(memory_space=pltpu.VMEM),
            pl.BlockSpec(memory_space=pltpu.VMEM),
        ],
    )(seq32, w_emb, w1, b1.reshape(1, H), w2, b2.reshape(1, D))


def _al8(x):
    return pl.multiple_of(x, 8)


def _piece(src_ref, dst_ref, buf, src_base, dst_base, off, rows):
    pltpu.sync_copy(src_ref.at[pl.ds(_al8(src_base + off), rows)], buf.at[pl.ds(0, rows)])
    pltpu.sync_copy(buf.at[pl.ds(0, rows)], dst_ref.at[pl.ds(_al8(dst_base + off), rows)])


def _copy_region(src_ref, dst_ref, buf, src_base, dst_base, count, max_count):
    nblk = (max_count + CAP - 1) // CAP
    for i in range(nblk):
        @pl.when(count >= (i + 1) * CAP)
        def _():
            _piece(src_ref, dst_ref, buf, src_base, dst_base, i * CAP, CAP)
    blkrows = (count // CAP) * CAP
    rem = count - blkrows
    for p in (64, 32, 16, 8):
        if p >= CAP or p > max_count:
            continue
        off = blkrows + jnp.bitwise_and(rem, jnp.int32(~(2 * p - 1)))

        @pl.when(jnp.bitwise_and(rem, jnp.int32(p)) != 0)
        def _():
            _piece(src_ref, dst_ref, buf, src_base, dst_base, off, p)


@functools.partial(
    pl.kernel,
    mesh=plsc.VectorSubcoreMesh(core_axis_name="c", subcore_axis_name="s"),
    out_type=jax.ShapeDtypeStruct((B * TOTAL, D), jnp.float32),
    scratch_types=[
        pltpu.VMEM((32,), jnp.int32),
        pltpu.VMEM((CAP, D), jnp.float32),
        pltpu.VMEM((8, D), jnp.float32),
        pltpu.VMEM((8, D), jnp.float32),
    ],
)
def _sc_union(inp_hbm, seq_hbm, aux_hbm, out_hbm, seq_v, buf, bin8, baux8):
    cid = lax.axis_index("c")
    sid = lax.axis_index("s")
    w = sid * 2 + cid
    b = w // WPB
    t0 = (w % WPB) * CHUNK

    pltpu.sync_copy(seq_hbm, seq_v)
    ln = seq_v[pl.ds(b, 16)][0]
    r = ln % 8
    g = ln - r
    g2 = g + V
    out_base = b * TOTAL

    k1 = jnp.clip(g - t0, 0, CHUNK)
    _copy_region(inp_hbm, out_hbm, buf, b * S + t0, out_base + t0, k1, CHUNK)

    va = g + jnp.where(r > 0, 8, 0)
    s2 = jnp.clip(va, t0, t0 + CHUNK)
    e2 = jnp.clip(g2, t0, t0 + CHUNK)
    _copy_region(aux_hbm.at[b], out_hbm, buf, s2 - g, out_base + s2, e2 - s2, V)

    @pl.when((r > 0) & (g2 >= t0) & (g2 < t0 + CHUNK))
    def _():
        _piece(aux_hbm.at[b], out_hbm, buf, jnp.int32(V), out_base + g2, 0, 8)

    z0 = jnp.clip(g2 + jnp.where(r > 0, 8, 0), t0, t0 + CHUNK)
    _copy_region(aux_hbm.at[b], out_hbm, buf, jnp.int32(V + 8), out_base + z0,
                 t0 + CHUNK - z0, CHUNK)

    @pl.when((r > 0) & (g >= t0) & (g < t0 + CHUNK))
    def _():
        pltpu.sync_copy(inp_hbm.at[pl.ds(_al8(b * S + g), 8)], bin8)
        pltpu.sync_copy(aux_hbm.at[b, pl.ds(0, 8)], baux8)

        def body(i, _):
            jj = i // 64
            c = (i % 64) * 16
            baux8[jj, pl.ds(c, 16)] = bin8[jj, pl.ds(c, 16)]
            return 0

        lax.fori_loop(0, r * 64, body, 0)
        pltpu.sync_copy(baux8, out_hbm.at[pl.ds(_al8(out_base + g), 8)])


def kernel(inputs, seq_len, W_emb, W1, b1, W2, b2):
    seq32 = seq_len.astype(jnp.int32)
    aux = _make_aux(seq32, W_emb, W1, b1, W2, b2)
    seq_pad = jnp.zeros((32,), jnp.int32).at[:B].set(seq32)
    out2d = _sc_union(inputs.reshape(B * S, D), seq_pad, aux)
    return out2d.reshape(B, TOTAL, D), seq_len + V
